# baseline (speedup 1.0000x reference)
import functools

import jax
import jax.numpy as jnp
from jax import lax
from jax.experimental import pallas as pl
from jax.experimental.pallas import tpu as pltpu
from jax.experimental.pallas import tpu_sc as plsc

D = 64
DP = 128
SCALE = 8.0
NC = 2
NS = 16
NW = NC * NS
NBUF = 4


@functools.lru_cache(maxsize=None)
def _make(S: int, L: int, V: int):
    LP = -(-L // 8) * 8
    SPW = S // NW
    G = SPW // 2
    CH = 2 * LP
    assert S % (2 * NW) == 0 and CH <= 128
    mesh = plsc.VectorSubcoreMesh(core_axis_name="c", subcore_axis_name="s")

    @functools.partial(
        pl.kernel,
        mesh=mesh,
        out_type=jax.ShapeDtypeStruct((S, LP, DP), jnp.float32),
        compiler_params=pltpu.CompilerParams(use_tc_tiling_on_sc=False),
        scratch_types=[
            pltpu.VMEM((G, CH), jnp.int32),
            *[pltpu.VMEM((CH, DP), jnp.float32) for _ in range(NBUF)],
            *[pltpu.SemaphoreType.DMA for _ in range(2 * NBUF)],
        ],
    )
    def emb(x_hbm, lut_hbm, out_hbm, idx_v, r0, r1, r2, r3,
            g0, g1, g2, g3, s0, s1, s2, s3):
        bufs = (r0, r1, r2, r3)
        gsem = (g0, g1, g2, g3)
        ssem = (s0, s1, s2, s3)
        wid = lax.axis_index("s") * NC + lax.axis_index("c")
        base = wid * SPW

        pltpu.sync_copy(x_hbm.at[wid], idx_v)

        def start_gather(g, b):
            pltpu.async_copy(lut_hbm.at[idx_v.at[g]], bufs[b], gsem[b])

        def wait_gather(g, b):
            pltpu.make_async_copy(lut_hbm.at[idx_v.at[g]], bufs[b],
                                  gsem[b]).wait()

        def scatters(g, b, op):
            for h in range(2):
                op(bufs[b].at[pl.ds(h * LP, L), pl.ds(0, D)],
                   out_hbm.at[base + 2 * g + h, pl.ds(0, L), pl.ds(0, D)],
                   ssem[b])

        def start_scatter(g, b):
            scatters(g, b, pltpu.async_copy)

        def wait_scatter(g, b):
            scatters(g, b, lambda s, d, m: pltpu.make_async_copy(s, d,
                                                                 m).wait())

        start_gather(0, 0)
        start_gather(1, 1)

        def body(i, carry):
            for b in range(NBUF):
                g = i * NBUF + b
                bn = (b + 2) % NBUF
                pl.when(g >= 2)(lambda: wait_scatter(g - 2, bn))
                pl.when(g + 2 < G)(lambda: start_gather(g + 2, bn))
                wait_gather(g, b)
                start_scatter(g, b)
            return carry

        lax.fori_loop(0, G // NBUF, body, 0)

        wait_scatter(G - 2, (G - 2) % NBUF)
        wait_scatter(G - 1, (G - 1) % NBUF)

    return emb


def kernel(x, lut):
    S, L = x.shape
    V = lut.shape[0]
    LP = -(-L // 8) * 8
    lutp = jnp.pad(lut * SCALE, ((0, 0), (0, DP - lut.shape[1])))
    x3 = jnp.pad(x.astype(jnp.int32).reshape(NW, S // NW, L),
                 ((0, 0), (0, 0), (0, LP - L)))
    x3 = x3.reshape(NW, S // NW // 2, 2 * LP)
    out = _make(S, L, V)(x3, lutp)
    return out[:, :L, :D]
